# baseline (speedup 1.0000x reference)
import functools

import jax
import jax.numpy as jnp
from jax import lax
from jax.experimental import pallas as pl
from jax.experimental.pallas import tpu as pltpu
from jax.experimental.pallas import tpu_sc as plsc

_NC, _NS = 2, 16
_NW = _NC * _NS
_B = 16384
_BPW = _B // _NW
_H = _BPW // 2
_D = 64
_P = 2 * _D
_V = 1000000
_EPS = 1e-10

_mesh = plsc.VectorSubcoreMesh(core_axis_name="c", subcore_axis_name="s")


@functools.partial(
    pl.kernel,
    out_type=[jax.ShapeDtypeStruct((_B, _P), jnp.float32),
              jax.ShapeDtypeStruct((_B, _P), jnp.float32)],
    mesh=_mesh,
    scratch_types=[
        pltpu.VMEM((_BPW,), jnp.int32),
        pltpu.VMEM((_BPW,), jnp.int32),
        pltpu.VMEM((_H, _P), jnp.float32),
        pltpu.VMEM((_H, _P), jnp.float32),
        pltpu.SemaphoreType.DMA,
        pltpu.SemaphoreType.DMA,
        pltpu.SemaphoreType.DMA,
    ],
)
def _gather_sc(b1_hbm, b2_hbm, emb2_hbm, o1_hbm, o2_hbm,
               idx1_v, idx2_v, rows1_v, rows2_v, sem1, sem2, wsem):
    wid = lax.axis_index("s") * _NC + lax.axis_index("c")
    base = wid * _BPW
    pltpu.sync_copy(b1_hbm.at[pl.ds(base, _BPW)], idx1_v)
    pltpu.sync_copy(b2_hbm.at[pl.ds(base, _BPW)], idx2_v)
    for h in range(_BPW // _H):
        g1 = [pltpu.async_copy(
            emb2_hbm.at[idx1_v.at[pl.ds(h * _H + j * 128, 128)]],
            rows1_v.at[pl.ds(j * 128, 128)], sem1)
            for j in range(_H // 128)]
        g2 = [pltpu.async_copy(
            emb2_hbm.at[idx2_v.at[pl.ds(h * _H + j * 128, 128)]],
            rows2_v.at[pl.ds(j * 128, 128)], sem2)
            for j in range(_H // 128)]
        for c in g1:
            c.wait()
        w1 = pltpu.async_copy(
            rows1_v, o1_hbm.at[pl.ds(base + h * _H, _H)], wsem)
        for c in g2:
            c.wait()
        w2 = pltpu.async_copy(
            rows2_v, o2_hbm.at[pl.ds(base + h * _H, _H)], wsem)
        w1.wait()
        w2.wait()


_BB = 2048
_NB = _B // _BB


def _loss_tc(e1_ref, e2_ref, h1_ref, h2_ref, rel_ref, gt_ref, w_ref,
             out_ref):
    i = pl.program_id(0)
    h1 = (h1_ref[...] == 1)
    h2 = (h2_ref[...] == 1)
    e1 = jnp.where(h1, e1_ref[:, _D:], e1_ref[:, :_D])
    e2 = jnp.where(h2, e2_ref[:, _D:], e2_ref[:, :_D])
    w = jax.nn.sigmoid(w_ref[...])
    p = e1 * e2
    s = lax.dot_general(p, w, (((1,), (1,)), ((), ())),
                        preferred_element_type=jnp.float32)
    col = lax.broadcasted_iota(jnp.int32, s.shape, 1)
    acc = jnp.sum(jnp.where(col == rel_ref[...], s, 0.0),
                  axis=1, keepdims=True)
    pred = jax.nn.sigmoid(acc)
    gt = gt_ref[...]
    loss = -(gt * jnp.log(pred + _EPS)
             + (1.0 - gt) * jnp.log(1.0 - pred + _EPS))
    part = jnp.sum(loss, keepdims=True).reshape(1, 1)

    @pl.when(i == 0)
    def _init():
        out_ref[...] = part

    @pl.when(i != 0)
    def _acc():
        out_ref[...] += part


def kernel(attr1, attr2, rel, ground_truth, embeddings, relation_embedding):
    a1 = attr1.astype(jnp.int32)
    a2 = attr2.astype(jnp.int32)
    emb2 = embeddings.reshape(_V // 2, _P)
    e1p, e2p = _gather_sc(a1 >> 1, a2 >> 1, emb2)
    h1 = (a1 & 1).reshape(_B, 1)
    h2 = (a2 & 1).reshape(_B, 1)
    rel2 = rel.astype(jnp.int32).reshape(_B, 1)
    gt2 = ground_truth.reshape(_B, 1)
    out = pl.pallas_call(
        _loss_tc,
        grid=(_NB,),
        in_specs=[
            pl.BlockSpec((_BB, _P), lambda i: (i, 0)),
            pl.BlockSpec((_BB, _P), lambda i: (i, 0)),
            pl.BlockSpec((_BB, 1), lambda i: (i, 0)),
            pl.BlockSpec((_BB, 1), lambda i: (i, 0)),
            pl.BlockSpec((_BB, 1), lambda i: (i, 0)),
            pl.BlockSpec((_BB, 1), lambda i: (i, 0)),
            pl.BlockSpec((_D, _D), lambda i: (0, 0)),
        ],
        out_specs=pl.BlockSpec((1, 1), lambda i: (0, 0)),
        out_shape=jax.ShapeDtypeStruct((1, 1), jnp.float32),
    )(e1p, e2p, h1, h2, rel2, gt2, relation_embedding)
    return out[0, 0]


# speedup vs baseline: 1.0001x; 1.0001x over previous
import functools

import jax
import jax.numpy as jnp
from jax import lax
from jax.experimental import pallas as pl
from jax.experimental.pallas import tpu as pltpu
from jax.experimental.pallas import tpu_sc as plsc

_NC, _NS = 2, 16
_NW = _NC * _NS
_B = 16384
_BPW = _B // _NW
_H = _BPW // 2
_D = 64
_P = 2 * _D
_V = 1000000
_EPS = 1e-10

_mesh = plsc.VectorSubcoreMesh(core_axis_name="c", subcore_axis_name="s")


@functools.partial(
    pl.kernel,
    out_type=[jax.ShapeDtypeStruct((_B, _P), jnp.float32),
              jax.ShapeDtypeStruct((_B, _P), jnp.float32)],
    mesh=_mesh,
    compiler_params=pltpu.CompilerParams(
        needs_layout_passes=False, use_tc_tiling_on_sc=True),
    scratch_types=[
        pltpu.VMEM((_BPW,), jnp.int32),
        pltpu.VMEM((_BPW,), jnp.int32),
        pltpu.VMEM((_H, _P), jnp.float32),
        pltpu.VMEM((_H, _P), jnp.float32),
        pltpu.SemaphoreType.DMA,
        pltpu.SemaphoreType.DMA,
        pltpu.SemaphoreType.DMA,
    ],
)
def _gather_sc(b1_hbm, b2_hbm, emb2_hbm, o1_hbm, o2_hbm,
               idx1_v, idx2_v, rows1_v, rows2_v, sem1, sem2, wsem):
    wid = lax.axis_index("s") * _NC + lax.axis_index("c")
    base = wid * _BPW
    pltpu.sync_copy(b1_hbm.at[pl.ds(base, _BPW)], idx1_v)
    pltpu.sync_copy(b2_hbm.at[pl.ds(base, _BPW)], idx2_v)
    for h in range(_BPW // _H):
        g1 = [pltpu.async_copy(
            emb2_hbm.at[idx1_v.at[pl.ds(h * _H + j * 128, 128)]],
            rows1_v.at[pl.ds(j * 128, 128)], sem1)
            for j in range(_H // 128)]
        g2 = [pltpu.async_copy(
            emb2_hbm.at[idx2_v.at[pl.ds(h * _H + j * 128, 128)]],
            rows2_v.at[pl.ds(j * 128, 128)], sem2)
            for j in range(_H // 128)]
        for c in g1:
            c.wait()
        w1 = pltpu.async_copy(
            rows1_v, o1_hbm.at[pl.ds(base + h * _H, _H)], wsem)
        for c in g2:
            c.wait()
        w2 = pltpu.async_copy(
            rows2_v, o2_hbm.at[pl.ds(base + h * _H, _H)], wsem)
        w1.wait()
        w2.wait()


_BB = 2048
_NB = _B // _BB


def _loss_tc(e1_ref, e2_ref, h1_ref, h2_ref, rel_ref, gt_ref, w_ref,
             out_ref):
    i = pl.program_id(0)
    h1 = (h1_ref[...] == 1)
    h2 = (h2_ref[...] == 1)
    e1 = jnp.where(h1, e1_ref[:, _D:], e1_ref[:, :_D])
    e2 = jnp.where(h2, e2_ref[:, _D:], e2_ref[:, :_D])
    w = jax.nn.sigmoid(w_ref[...])
    p = e1 * e2
    s = lax.dot_general(p, w, (((1,), (1,)), ((), ())),
                        preferred_element_type=jnp.float32)
    col = lax.broadcasted_iota(jnp.int32, s.shape, 1)
    acc = jnp.sum(jnp.where(col == rel_ref[...], s, 0.0),
                  axis=1, keepdims=True)
    pred = jax.nn.sigmoid(acc)
    gt = gt_ref[...]
    loss = -(gt * jnp.log(pred + _EPS)
             + (1.0 - gt) * jnp.log(1.0 - pred + _EPS))
    part = jnp.sum(loss, keepdims=True).reshape(1, 1)

    @pl.when(i == 0)
    def _init():
        out_ref[...] = part

    @pl.when(i != 0)
    def _acc():
        out_ref[...] += part


def kernel(attr1, attr2, rel, ground_truth, embeddings, relation_embedding):
    a1 = attr1.astype(jnp.int32)
    a2 = attr2.astype(jnp.int32)
    emb2 = embeddings.reshape(_V // 2, _P)
    e1p, e2p = _gather_sc(a1 >> 1, a2 >> 1, emb2)
    h1 = (a1 & 1).reshape(_B, 1)
    h2 = (a2 & 1).reshape(_B, 1)
    rel2 = rel.astype(jnp.int32).reshape(_B, 1)
    gt2 = ground_truth.reshape(_B, 1)
    out = pl.pallas_call(
        _loss_tc,
        grid=(_NB,),
        in_specs=[
            pl.BlockSpec((_BB, _P), lambda i: (i, 0)),
            pl.BlockSpec((_BB, _P), lambda i: (i, 0)),
            pl.BlockSpec((_BB, 1), lambda i: (i, 0)),
            pl.BlockSpec((_BB, 1), lambda i: (i, 0)),
            pl.BlockSpec((_BB, 1), lambda i: (i, 0)),
            pl.BlockSpec((_BB, 1), lambda i: (i, 0)),
            pl.BlockSpec((_D, _D), lambda i: (0, 0)),
        ],
        out_specs=pl.BlockSpec((1, 1), lambda i: (0, 0)),
        out_shape=jax.ShapeDtypeStruct((1, 1), jnp.float32),
    )(e1p, e2p, h1, h2, rel2, gt2, relation_embedding)
    return out[0, 0]


# speedup vs baseline: 1.0023x; 1.0022x over previous
import functools

import jax
import jax.numpy as jnp
from jax import lax
from jax.experimental import pallas as pl
from jax.experimental.pallas import tpu as pltpu
from jax.experimental.pallas import tpu_sc as plsc

_NC, _NS = 2, 16
_NW = _NC * _NS
_B = 16384
_BPW = _B // _NW
_H = _BPW // 2
_D = 64
_P = 2 * _D
_V = 1000000
_EPS = 1e-10

_mesh = plsc.VectorSubcoreMesh(core_axis_name="c", subcore_axis_name="s")


@functools.partial(
    pl.kernel,
    out_type=[jax.ShapeDtypeStruct((_B, _P), jnp.float32),
              jax.ShapeDtypeStruct((_B, _P), jnp.float32)],
    mesh=_mesh,
    compiler_params=pltpu.CompilerParams(
        needs_layout_passes=False, use_tc_tiling_on_sc=True),
    scratch_types=[
        pltpu.VMEM((_BPW,), jnp.int32),
        pltpu.VMEM((_BPW,), jnp.int32),
        pltpu.VMEM((_H, _P), jnp.float32),
        pltpu.VMEM((_H, _P), jnp.float32),
        pltpu.SemaphoreType.DMA,
        pltpu.SemaphoreType.DMA,
        pltpu.SemaphoreType.DMA,
    ],
)
def _gather_sc(b1_hbm, b2_hbm, emb2_hbm, o1_hbm, o2_hbm,
               idx1_v, idx2_v, rows1_v, rows2_v, sem1, sem2, wsem):
    wid = lax.axis_index("s") * _NC + lax.axis_index("c")
    base = wid * _BPW
    pltpu.sync_copy(b1_hbm.at[pl.ds(base, _BPW)], idx1_v)
    pltpu.sync_copy(b2_hbm.at[pl.ds(base, _BPW)], idx2_v)
    for h in range(_BPW // _H):
        g1 = [pltpu.async_copy(
            emb2_hbm.at[idx1_v.at[pl.ds(h * _H + j * 128, 128)]],
            rows1_v.at[pl.ds(j * 128, 128)], sem1)
            for j in range(_H // 128)]
        g2 = [pltpu.async_copy(
            emb2_hbm.at[idx2_v.at[pl.ds(h * _H + j * 128, 128)]],
            rows2_v.at[pl.ds(j * 128, 128)], sem2)
            for j in range(_H // 128)]
        for c in g1:
            c.wait()
        w1 = pltpu.async_copy(
            rows1_v, o1_hbm.at[pl.ds(base + h * _H, _H)], wsem)
        for c in g2:
            c.wait()
        w2 = pltpu.async_copy(
            rows2_v, o2_hbm.at[pl.ds(base + h * _H, _H)], wsem)
        w1.wait()
        w2.wait()


_RB = 8192
_NRB = _V // _RB


def _pack_tc(src_ref, dst_ref):
    dst_ref[...] = src_ref[...].reshape(_RB // 2, _P)


def _pack_pairs(embeddings):
    return pl.pallas_call(
        _pack_tc,
        grid=(_NRB,),
        in_specs=[pl.BlockSpec((_RB, _D), lambda i: (i, 0))],
        out_specs=pl.BlockSpec((_RB // 2, _P), lambda i: (i, 0)),
        out_shape=jax.ShapeDtypeStruct((_V // 2, _P), jnp.float32),
    )(embeddings)


_BB = 2048
_NB = _B // _BB


def _loss_tc(e1_ref, e2_ref, h1_ref, h2_ref, rel_ref, gt_ref, w_ref,
             out_ref):
    i = pl.program_id(0)
    h1 = (h1_ref[...] == 1)
    h2 = (h2_ref[...] == 1)
    e1 = jnp.where(h1, e1_ref[:, _D:], e1_ref[:, :_D])
    e2 = jnp.where(h2, e2_ref[:, _D:], e2_ref[:, :_D])
    w = jax.nn.sigmoid(w_ref[...])
    p = e1 * e2
    s = lax.dot_general(p, w, (((1,), (1,)), ((), ())),
                        preferred_element_type=jnp.float32)
    col = lax.broadcasted_iota(jnp.int32, s.shape, 1)
    acc = jnp.sum(jnp.where(col == rel_ref[...], s, 0.0),
                  axis=1, keepdims=True)
    pred = jax.nn.sigmoid(acc)
    gt = gt_ref[...]
    loss = -(gt * jnp.log(pred + _EPS)
             + (1.0 - gt) * jnp.log(1.0 - pred + _EPS))
    part = jnp.sum(loss, keepdims=True).reshape(1, 1)

    @pl.when(i == 0)
    def _init():
        out_ref[...] = part

    @pl.when(i != 0)
    def _acc():
        out_ref[...] += part


def kernel(attr1, attr2, rel, ground_truth, embeddings, relation_embedding):
    a1 = attr1.astype(jnp.int32)
    a2 = attr2.astype(jnp.int32)
    emb2 = embeddings.reshape(_V // 2, _P)
    e1p, e2p = _gather_sc(a1 >> 1, a2 >> 1, emb2)
    h1 = (a1 & 1).reshape(_B, 1)
    h2 = (a2 & 1).reshape(_B, 1)
    rel2 = rel.astype(jnp.int32).reshape(_B, 1)
    gt2 = ground_truth.reshape(_B, 1)
    out = pl.pallas_call(
        _loss_tc,
        grid=(_NB,),
        in_specs=[
            pl.BlockSpec((_BB, _P), lambda i: (i, 0)),
            pl.BlockSpec((_BB, _P), lambda i: (i, 0)),
            pl.BlockSpec((_BB, 1), lambda i: (i, 0)),
            pl.BlockSpec((_BB, 1), lambda i: (i, 0)),
            pl.BlockSpec((_BB, 1), lambda i: (i, 0)),
            pl.BlockSpec((_BB, 1), lambda i: (i, 0)),
            pl.BlockSpec((_D, _D), lambda i: (0, 0)),
        ],
        out_specs=pl.BlockSpec((1, 1), lambda i: (0, 0)),
        out_shape=jax.ShapeDtypeStruct((1, 1), jnp.float32),
    )(e1p, e2p, h1, h2, rel2, gt2, relation_embedding)
    return out[0, 0]
